# baseline (speedup 1.0000x reference)
import jax
import jax.numpy as jnp
from jax.experimental import pallas as pl

_SEQ = 512
_ROWS_PER_BLOCK = 16


def _copy_kernel(s_ref, out_ref):
    base = pl.program_id(0)
    for r in range(_ROWS_PER_BLOCK):
        k = (511 - r) % 8
        off = ((511 - r) // 8 - base * (_ROWS_PER_BLOCK // 8)) * 8
        out_ref[r] = s_ref[k, pl.ds(off, _SEQ), :]


def kernel(hidden_states, positional_params):
    seq = hidden_states.shape[1]
    hidden = positional_params.shape[1]
    shifted = jnp.stack(
        [jax.lax.dynamic_slice_in_dim(positional_params, k, 2 * seq - 8, axis=0)
         for k in range(8)]
    )
    grid = (seq // _ROWS_PER_BLOCK,)
    return pl.pallas_call(
        _copy_kernel,
        grid=grid,
        in_specs=[
            pl.BlockSpec(shifted.shape, lambda i: (0, 0, 0)),
        ],
        out_specs=pl.BlockSpec(
            (_ROWS_PER_BLOCK, seq, hidden), lambda i: (i, 0, 0)
        ),
        out_shape=jax.ShapeDtypeStruct((seq, seq, hidden), positional_params.dtype),
    )(shifted)


# speedup vs baseline: 1.0149x; 1.0149x over previous
import jax
import jax.numpy as jnp
from jax import lax
from jax.experimental import pallas as pl
from jax.experimental.pallas import tpu as pltpu

_SEQ = 512


def _dma_kernel(s_ref, out_ref, sem):
    for r in range(8):
        k = 7 - r

        def issue(b, carry, k=k, r=r):
            off = pl.multiple_of((63 - b) * 8, 8)
            pltpu.make_async_copy(
                s_ref.at[k, pl.ds(off, _SEQ), :], out_ref.at[8 * b + r], sem
            ).start()
            return carry

        lax.fori_loop(0, _SEQ // 8, issue, 0)

    def drain(b, carry):
        pltpu.make_async_copy(
            s_ref.at[0, pl.ds(0, _SEQ), :], out_ref.at[0], sem
        ).wait()
        return carry

    lax.fori_loop(0, _SEQ, drain, 0)


def kernel(hidden_states, positional_params):
    seq = hidden_states.shape[1]
    hidden = positional_params.shape[1]
    shifted = jnp.stack(
        [jax.lax.dynamic_slice_in_dim(positional_params, k, 2 * seq - 8, axis=0)
         for k in range(8)]
    )
    return pl.pallas_call(
        _dma_kernel,
        in_specs=[pl.BlockSpec(memory_space=pltpu.MemorySpace.VMEM)],
        out_specs=pl.BlockSpec(memory_space=pltpu.MemorySpace.HBM),
        out_shape=jax.ShapeDtypeStruct((seq, seq, hidden), positional_params.dtype),
        scratch_shapes=[pltpu.SemaphoreType.DMA],
    )(shifted)
